# Initial kernel scaffold; baseline (speedup 1.0000x reference)
import jax
import jax.numpy as jnp
from jax.experimental import pallas as pl


def kernel(x_cat, x_cont, tables, W1, b1, W2, b2, W3, b3):
    raise NotImplementedError("write your pallas kernel here")


# speedup vs baseline: 1.5092x; 1.5092x over previous
import functools

import jax
import jax.numpy as jnp
from jax import lax
from jax.experimental import pallas as pl
from jax.experimental.pallas import tpu as pltpu
from jax.experimental.pallas import tpu_sc as plsc

_N_FIELDS = 26
_VOCAB = 100000
_EMB_DIM = 16
_BATCH = 16384
_N_CONT = 13
_N_EMB = _N_FIELDS * _EMB_DIM

_NC, _NS = 2, 16
_NW = _NC * _NS
_B_PER_W = _BATCH // _NW
_N_FLD_PAD = 32
_N_EMB_PAD = _N_FLD_PAD * _EMB_DIM
_CHUNK = 128
_N_CHUNKS = _B_PER_W // _CHUNK


@functools.partial(
    pl.kernel,
    out_type=jax.ShapeDtypeStruct((_BATCH * _N_FLD_PAD, _EMB_DIM),
                                  jnp.float32),
    mesh=plsc.VectorSubcoreMesh(core_axis_name="c", subcore_axis_name="s"),
    compiler_params=pltpu.CompilerParams(use_tc_tiling_on_sc=False),
    scratch_types=[
        pltpu.VMEM((_CHUNK * _N_FLD_PAD,), jnp.int32),
        pltpu.VMEM((_CHUNK * _N_FLD_PAD, _EMB_DIM), jnp.float32),
        pltpu.SemaphoreType.DMA,
    ],
)
def _gather_sc(idx_hbm, tables_hbm, out_hbm, idx_v, rows_v, sem):
    wid = lax.axis_index("s") * _NC + lax.axis_index("c")
    base = wid * _B_PER_W
    for c in range(_N_CHUNKS):
        row0 = (base + c * _CHUNK) * _N_FLD_PAD
        pltpu.sync_copy(
            idx_hbm.at[pl.ds(row0, _CHUNK * _N_FLD_PAD)], idx_v)
        pltpu.async_copy(tables_hbm.at[idx_v], rows_v, sem).wait()
        pltpu.sync_copy(rows_v, out_hbm.at[pl.ds(row0, _CHUNK * _N_FLD_PAD), :])


def _mlp_body(emb_ref, xc_ref, w1e_ref, w1c_ref, b1_ref, w2_ref, b2_ref,
              w3_ref, b3_ref, out_ref):
    cdims = (((1,), (1,)), ((), ()))
    h = lax.dot_general(emb_ref[...], w1e_ref[...], cdims,
                        preferred_element_type=jnp.float32)
    h += lax.dot_general(xc_ref[...], w1c_ref[...], cdims,
                         preferred_element_type=jnp.float32)
    h = jnp.maximum(h + b1_ref[...], 0.0)
    h = jnp.maximum(
        lax.dot_general(h, w2_ref[...], cdims,
                        preferred_element_type=jnp.float32) + b2_ref[...], 0.0)
    y = lax.dot_general(h, w3_ref[...], cdims,
                        preferred_element_type=jnp.float32)
    out_ref[...] = y[:, 0:1] + b3_ref[0, 0]


_MLP_BLK = 2048


def _mlp_tc(emb, x_cont, w1e, w1c, b1, w2, b2, w3, b3):
    grid = (_BATCH // _MLP_BLK,)
    full = lambda shape: pl.BlockSpec(shape, lambda i: (0, 0))
    return pl.pallas_call(
        _mlp_body,
        grid=grid,
        in_specs=[
            pl.BlockSpec((_MLP_BLK, _N_EMB_PAD), lambda i: (i, 0)),
            pl.BlockSpec((_MLP_BLK, _N_CONT), lambda i: (i, 0)),
            full(w1e.shape), full(w1c.shape), full(b1.shape),
            full(w2.shape), full(b2.shape), full(w3.shape),
            pl.BlockSpec(memory_space=pltpu.SMEM),
        ],
        out_specs=pl.BlockSpec((_MLP_BLK, 1), lambda i: (i, 0)),
        out_shape=jax.ShapeDtypeStruct((_BATCH, 1), jnp.float32),
    )(emb, x_cont, w1e, w1c, b1, w2, b2, w3, b3)


def kernel(x_cat, x_cont, tables, W1, b1, W2, b2, W3, b3):
    xc = jnp.asarray(x_cat, jnp.int32)
    offs = jnp.arange(_N_FIELDS, dtype=jnp.int32) * _VOCAB
    idx32 = jnp.concatenate(
        [xc + offs, jnp.zeros((_BATCH, _N_FLD_PAD - _N_FIELDS), jnp.int32)],
        axis=1)
    idx_flat = idx32.reshape(-1)
    rows = _gather_sc(idx_flat, tables.reshape(_N_FIELDS * _VOCAB, _EMB_DIM))
    emb = rows.reshape(_BATCH, _N_EMB_PAD)
    w1e = jnp.pad(W1[:, :_N_EMB], ((0, 0), (0, _N_EMB_PAD - _N_EMB)))
    w1c = W1[:, _N_EMB:]
    w3 = jnp.pad(W3, ((0, 7), (0, 0)))
    return _mlp_tc(emb, x_cont, w1e, w1c, b1.reshape(1, -1), W2,
                   b2.reshape(1, -1), w3, b3.reshape(1, -1))
